# baseline (speedup 1.0000x reference)
import functools

import jax
import jax.numpy as jnp
from jax import lax
from jax.experimental import pallas as pl
from jax.experimental.pallas import tpu as pltpu
from jax.experimental.pallas import tpu_sc as plsc

N = 10000
E = 320000
D = 128

NC = 2
NS = 16
NW = NC * NS
EPW = E // NW
CH = 80
NCHUNK = EPW // CH
NBUF = 12
LOOK = NBUF // 2
NMAIN = (NCHUNK // NBUF) * NBUF

ROWS_A = 624
ROWS_LAST = N - (NS - 1) * ROWS_A
DEG_P = 10240


EBLK = E // 10


def _proj_body(x_ref, w1_ref, e_ref, y_ref, src_ref, dst_ref):
    y_ref[...] = jnp.dot(x_ref[...], w1_ref[...],
                         preferred_element_type=jnp.float32).astype(jnp.bfloat16)
    i = pl.program_id(0)
    sl = pl.ds(i * EBLK, EBLK)
    src_ref[sl] = e_ref[0, sl]
    dst_ref[sl] = e_ref[1, sl]


def _project(x, w1t, edge_index, blk=1000):
    return pl.pallas_call(
        _proj_body,
        grid=(N // blk,),
        in_specs=[
            pl.BlockSpec((blk, D), lambda i: (i, 0)),
            pl.BlockSpec((D, D), lambda i: (0, 0)),
            pl.BlockSpec((2, E), lambda i: (0, 0)),
        ],
        out_specs=[
            pl.BlockSpec((blk, D), lambda i: (i, 0)),
            pl.BlockSpec((E,), lambda i: (0,)),
            pl.BlockSpec((E,), lambda i: (0,)),
        ],
        out_shape=[
            jax.ShapeDtypeStruct((N, D), jnp.bfloat16),
            jax.ShapeDtypeStruct((E,), jnp.int32),
            jax.ShapeDtypeStruct((E,), jnp.int32),
        ],
    )(x, w1t, edge_index)


def _zproj_body(x_ref, w2_ref, z_ref):
    z_ref[...] = jnp.dot(x_ref[...], w2_ref[...],
                         preferred_element_type=jnp.float32)


def _zproject(x, w2t, blk=1000):
    return pl.pallas_call(
        _zproj_body,
        grid=(N // blk,),
        in_specs=[
            pl.BlockSpec((blk, D), lambda i: (i, 0)),
            pl.BlockSpec((D, D), lambda i: (0, 0)),
        ],
        out_specs=pl.BlockSpec((blk, D), lambda i: (i, 0)),
        out_shape=jax.ShapeDtypeStruct((N, D), jnp.float32),
    )(x, w2t)


def _seg_body(y_hbm, z_hbm, z1_hbm, src_hbm, dst_hbm, out_hbm, deg_hbm,
              sidx, didx, ones_v, rows, acc, dacc, gsems, ssems, dsem):
    c = lax.axis_index("c")
    s = lax.axis_index("s")
    wid = s * NC + c
    base = wid * EPW

    r0 = s * ROWS_A

    @pl.when(s < NS - 1)
    def _():
        pltpu.sync_copy(z_hbm.at[pl.ds(0, ROWS_A)], acc.at[pl.ds(r0, ROWS_A)])

    @pl.when(s == NS - 1)
    def _():
        pltpu.sync_copy(z_hbm, acc.at[pl.ds(r0, ROWS_LAST)])

    pltpu.sync_copy(z1_hbm, dacc.at[pl.ds(s * ROWS_LAST, ROWS_LAST)])

    ov = jnp.full((16,), 1.0, jnp.float32)
    for k in range(CH // 16):
        ones_v[pl.ds(k * 16, 16)] = ov
    ones_v[pl.ds(CH - 16, 16)] = ov

    pltpu.sync_copy(src_hbm.at[pl.ds(base, EPW)], sidx)
    pltpu.sync_copy(dst_hbm.at[pl.ds(base, EPW)], didx)

    plsc.subcore_barrier()

    def wait_rows(sem):
        pltpu.make_async_copy(y_hbm.at[pl.ds(0, CH)], rows[0], sem).wait()

    def issue_gather(i, b):
        pltpu.async_copy(y_hbm.at[sidx.at[pl.ds(i * CH, CH)]], rows[b], gsems[b])

    for b in range(LOOK):
        issue_gather(b, b)

    def body(i, b):
        wait_rows(gsems[b])
        didx_i = didx.at[pl.ds(i * CH, CH)]
        pltpu.async_copy(rows[b], acc.at[didx_i], ssems[b], add=True)
        pltpu.async_copy(ones_v, dacc.at[didx_i], dsem, add=True)
        j = i + LOOK
        bj = (b + LOOK) % NBUF

        @pl.when(jnp.logical_and(j < NCHUNK, i >= LOOK))
        def _():
            wait_rows(ssems[bj])
            issue_gather(j, bj)

        @pl.when(jnp.logical_and(j < NCHUNK, i < LOOK))
        def _():
            issue_gather(j, bj)

    def group(g, carry):
        for b in range(NBUF):
            body(g * NBUF + b, b)
        return carry

    lax.fori_loop(0, NMAIN // NBUF, group, 0)

    for i in range(NMAIN, NCHUNK):
        b = i % NBUF
        wait_rows(gsems[b])
        didx_t = didx.at[pl.ds(i * CH, CH)]
        pltpu.async_copy(rows[b], acc.at[didx_t], ssems[b], add=True)
        pltpu.async_copy(ones_v, dacc.at[didx_t], dsem, add=True)

    for b in range(NBUF):
        wait_rows(ssems[b])
    pltpu.make_async_copy(src_hbm.at[pl.ds(0, EPW)], sidx, dsem).wait()

    plsc.subcore_barrier()

    @pl.when(s < NS - 1)
    def _():
        pltpu.sync_copy(acc.at[pl.ds(r0, ROWS_A)], out_hbm.at[c, pl.ds(r0, ROWS_A)])

    @pl.when(s == NS - 1)
    def _():
        pltpu.sync_copy(acc.at[pl.ds(r0, ROWS_LAST)],
                        out_hbm.at[c, pl.ds(r0, ROWS_LAST)])

    pltpu.sync_copy(dacc.at[pl.ds(s * ROWS_LAST, ROWS_LAST)],
                    deg_hbm.at[c, pl.ds(s * ROWS_LAST, ROWS_LAST)])


def _segment_sum(y, zeros, zeros1, src, dst):
    mesh = plsc.VectorSubcoreMesh(core_axis_name="c", subcore_axis_name="s")
    f = functools.partial(
        pl.kernel,
        out_type=[
            jax.ShapeDtypeStruct((NC, N, D), jnp.bfloat16),
            jax.ShapeDtypeStruct((NC, DEG_P), jnp.float32),
        ],
        mesh=mesh,
        scratch_types=[
            pltpu.VMEM((EPW,), jnp.int32),
            pltpu.VMEM((EPW,), jnp.int32),
            pltpu.VMEM((CH,), jnp.float32),
            [pltpu.VMEM((CH, D), jnp.bfloat16) for _ in range(NBUF)],
            pltpu.VMEM_SHARED((N, D), jnp.bfloat16),
            pltpu.VMEM_SHARED((DEG_P,), jnp.float32),
            [pltpu.SemaphoreType.DMA for _ in range(NBUF)],
            [pltpu.SemaphoreType.DMA for _ in range(NBUF)],
            pltpu.SemaphoreType.DMA,
        ],
        compiler_params=pltpu.CompilerParams(use_tc_tiling_on_sc=False),
    )(_seg_body)
    return f(y, zeros, zeros1, src, dst)


def _final_body(s_ref, deg_ref, z_ref, o_ref):
    stot = (s_ref[0].astype(jnp.float32)
            + s_ref[1].astype(jnp.float32))
    deg = deg_ref[0, :N] + deg_ref[1, :N]
    deg = jnp.maximum(deg, 1.0)
    o_ref[...] = stot / deg[:, None] + z_ref[...]


def _finalize(s, degs, z):
    return pl.pallas_call(
        _final_body,
        out_shape=jax.ShapeDtypeStruct((N, D), jnp.float32),
    )(s, degs, z)


def kernel(in_features, edge_index, W1, W2, layer_id):
    y, src, dst = _project(in_features, W1.T, edge_index)
    z = _zproject(in_features, W2.T)
    zeros = jnp.zeros((ROWS_LAST, D), jnp.bfloat16)
    zeros1 = jnp.zeros((ROWS_LAST,), jnp.float32)
    s, degs = _segment_sum(y, zeros, zeros1, src, dst)
    return _finalize(s, degs, z)


# speedup vs baseline: 1.0126x; 1.0126x over previous
import functools

import jax
import jax.numpy as jnp
from jax import lax
from jax.experimental import pallas as pl
from jax.experimental.pallas import tpu as pltpu
from jax.experimental.pallas import tpu_sc as plsc

N = 10000
E = 320000
D = 128

NC = 2
NS = 16
NW = NC * NS
EPW = E // NW
CH = 80
NCHUNK = EPW // CH
NBUF = 13
LOOK = 8
NMAIN = (NCHUNK // NBUF) * NBUF

ROWS_A = 624
ROWS_LAST = N - (NS - 1) * ROWS_A
DEG_P = 10240


EBLK = E // 10


def _proj_body(x_ref, w1_ref, e_ref, y_ref, src_ref, dst_ref):
    y_ref[...] = jnp.dot(x_ref[...], w1_ref[...],
                         preferred_element_type=jnp.float32).astype(jnp.bfloat16)
    i = pl.program_id(0)
    sl = pl.ds(i * EBLK, EBLK)
    src_ref[sl] = e_ref[0, sl]
    dst_ref[sl] = e_ref[1, sl]


def _project(x, w1t, edge_index, blk=1000):
    return pl.pallas_call(
        _proj_body,
        grid=(N // blk,),
        in_specs=[
            pl.BlockSpec((blk, D), lambda i: (i, 0)),
            pl.BlockSpec((D, D), lambda i: (0, 0)),
            pl.BlockSpec((2, E), lambda i: (0, 0)),
        ],
        out_specs=[
            pl.BlockSpec((blk, D), lambda i: (i, 0)),
            pl.BlockSpec((E,), lambda i: (0,)),
            pl.BlockSpec((E,), lambda i: (0,)),
        ],
        out_shape=[
            jax.ShapeDtypeStruct((N, D), jnp.bfloat16),
            jax.ShapeDtypeStruct((E,), jnp.int32),
            jax.ShapeDtypeStruct((E,), jnp.int32),
        ],
    )(x, w1t, edge_index)


def _zproj_body(x_ref, w2_ref, z_ref):
    z_ref[...] = jnp.dot(x_ref[...], w2_ref[...],
                         preferred_element_type=jnp.float32)


def _zproject(x, w2t, blk=1000):
    return pl.pallas_call(
        _zproj_body,
        grid=(N // blk,),
        in_specs=[
            pl.BlockSpec((blk, D), lambda i: (i, 0)),
            pl.BlockSpec((D, D), lambda i: (0, 0)),
        ],
        out_specs=pl.BlockSpec((blk, D), lambda i: (i, 0)),
        out_shape=jax.ShapeDtypeStruct((N, D), jnp.float32),
    )(x, w2t)


def _seg_body(y_hbm, z_hbm, z1_hbm, src_hbm, dst_hbm, out_hbm, deg_hbm,
              sidx, didx, ones_v, rows, acc, dacc, gsems, ssems, dsem):
    c = lax.axis_index("c")
    s = lax.axis_index("s")
    wid = s * NC + c
    base = wid * EPW

    r0 = s * ROWS_A

    @pl.when(s < NS - 1)
    def _():
        pltpu.sync_copy(z_hbm.at[pl.ds(0, ROWS_A)], acc.at[pl.ds(r0, ROWS_A)])

    @pl.when(s == NS - 1)
    def _():
        pltpu.sync_copy(z_hbm, acc.at[pl.ds(r0, ROWS_LAST)])

    pltpu.sync_copy(z1_hbm, dacc.at[pl.ds(s * ROWS_LAST, ROWS_LAST)])

    ov = jnp.full((16,), 1.0, jnp.float32)
    for k in range(CH // 16):
        ones_v[pl.ds(k * 16, 16)] = ov
    ones_v[pl.ds(CH - 16, 16)] = ov

    pltpu.sync_copy(src_hbm.at[pl.ds(base, EPW)], sidx)
    pltpu.sync_copy(dst_hbm.at[pl.ds(base, EPW)], didx)

    plsc.subcore_barrier()

    def wait_rows(sem):
        pltpu.make_async_copy(y_hbm.at[pl.ds(0, CH)], rows[0], sem).wait()

    def issue_gather(i, b):
        pltpu.async_copy(y_hbm.at[sidx.at[pl.ds(i * CH, CH)]], rows[b], gsems[b])

    for b in range(LOOK):
        issue_gather(b, b)

    def body(i, b):
        wait_rows(gsems[b])
        didx_i = didx.at[pl.ds(i * CH, CH)]
        pltpu.async_copy(rows[b], acc.at[didx_i], ssems[b], add=True)
        pltpu.async_copy(ones_v, dacc.at[didx_i], dsem, add=True)
        j = i + LOOK
        bj = (b + LOOK) % NBUF

        @pl.when(jnp.logical_and(j < NCHUNK, i >= NBUF - LOOK))
        def _():
            wait_rows(ssems[bj])
            issue_gather(j, bj)

        @pl.when(jnp.logical_and(j < NCHUNK, i < NBUF - LOOK))
        def _():
            issue_gather(j, bj)

    def group(g, carry):
        for b in range(NBUF):
            body(g * NBUF + b, b)
        return carry

    lax.fori_loop(0, NMAIN // NBUF, group, 0)

    for i in range(NMAIN, NCHUNK):
        b = i % NBUF
        wait_rows(gsems[b])
        didx_t = didx.at[pl.ds(i * CH, CH)]
        pltpu.async_copy(rows[b], acc.at[didx_t], ssems[b], add=True)
        pltpu.async_copy(ones_v, dacc.at[didx_t], dsem, add=True)

    for b in range(NBUF):
        wait_rows(ssems[b])
    pltpu.make_async_copy(src_hbm.at[pl.ds(0, EPW)], sidx, dsem).wait()

    plsc.subcore_barrier()

    @pl.when(s < NS - 1)
    def _():
        pltpu.sync_copy(acc.at[pl.ds(r0, ROWS_A)], out_hbm.at[c, pl.ds(r0, ROWS_A)])

    @pl.when(s == NS - 1)
    def _():
        pltpu.sync_copy(acc.at[pl.ds(r0, ROWS_LAST)],
                        out_hbm.at[c, pl.ds(r0, ROWS_LAST)])

    pltpu.sync_copy(dacc.at[pl.ds(s * ROWS_LAST, ROWS_LAST)],
                    deg_hbm.at[c, pl.ds(s * ROWS_LAST, ROWS_LAST)])


def _segment_sum(y, zeros, zeros1, src, dst):
    mesh = plsc.VectorSubcoreMesh(core_axis_name="c", subcore_axis_name="s")
    f = functools.partial(
        pl.kernel,
        out_type=[
            jax.ShapeDtypeStruct((NC, N, D), jnp.bfloat16),
            jax.ShapeDtypeStruct((NC, DEG_P), jnp.float32),
        ],
        mesh=mesh,
        scratch_types=[
            pltpu.VMEM((EPW,), jnp.int32),
            pltpu.VMEM((EPW,), jnp.int32),
            pltpu.VMEM((CH,), jnp.float32),
            [pltpu.VMEM((CH, D), jnp.bfloat16) for _ in range(NBUF)],
            pltpu.VMEM_SHARED((N, D), jnp.bfloat16),
            pltpu.VMEM_SHARED((DEG_P,), jnp.float32),
            [pltpu.SemaphoreType.DMA for _ in range(NBUF)],
            [pltpu.SemaphoreType.DMA for _ in range(NBUF)],
            pltpu.SemaphoreType.DMA,
        ],
        compiler_params=pltpu.CompilerParams(use_tc_tiling_on_sc=False),
    )(_seg_body)
    return f(y, zeros, zeros1, src, dst)


def _final_body(s_ref, deg_ref, z_ref, o_ref):
    stot = (s_ref[0].astype(jnp.float32)
            + s_ref[1].astype(jnp.float32))
    deg = deg_ref[0, :N] + deg_ref[1, :N]
    deg = jnp.maximum(deg, 1.0)
    o_ref[...] = stot / deg[:, None] + z_ref[...]


def _finalize(s, degs, z):
    return pl.pallas_call(
        _final_body,
        out_shape=jax.ShapeDtypeStruct((N, D), jnp.float32),
    )(s, degs, z)


def kernel(in_features, edge_index, W1, W2, layer_id):
    y, src, dst = _project(in_features, W1.T, edge_index)
    z = _zproject(in_features, W2.T)
    zeros = jnp.zeros((ROWS_LAST, D), jnp.bfloat16)
    zeros1 = jnp.zeros((ROWS_LAST,), jnp.float32)
    s, degs = _segment_sum(y, zeros, zeros1, src, dst)
    return _finalize(s, degs, z)


# speedup vs baseline: 1.0128x; 1.0003x over previous
import functools

import jax
import jax.numpy as jnp
from jax import lax
from jax.experimental import pallas as pl
from jax.experimental.pallas import tpu as pltpu
from jax.experimental.pallas import tpu_sc as plsc

N = 10000
E = 320000
D = 128

NC = 2
NS = 16
NW = NC * NS
EPW = E // NW
CH = 80
NCHUNK = EPW // CH
NBUF = 13
LOOK = 9
NMAIN = (NCHUNK // NBUF) * NBUF

ROWS_A = 624
ROWS_LAST = N - (NS - 1) * ROWS_A
DEG_P = 10240


EBLK = E // 10


def _proj_body(x_ref, w1_ref, e_ref, y_ref, src_ref, dst_ref):
    y_ref[...] = jnp.dot(x_ref[...], w1_ref[...],
                         preferred_element_type=jnp.float32).astype(jnp.bfloat16)
    i = pl.program_id(0)
    sl = pl.ds(i * EBLK, EBLK)
    src_ref[sl] = e_ref[0, sl]
    dst_ref[sl] = e_ref[1, sl]


def _project(x, w1t, edge_index, blk=1000):
    return pl.pallas_call(
        _proj_body,
        grid=(N // blk,),
        in_specs=[
            pl.BlockSpec((blk, D), lambda i: (i, 0)),
            pl.BlockSpec((D, D), lambda i: (0, 0)),
            pl.BlockSpec((2, E), lambda i: (0, 0)),
        ],
        out_specs=[
            pl.BlockSpec((blk, D), lambda i: (i, 0)),
            pl.BlockSpec((E,), lambda i: (0,)),
            pl.BlockSpec((E,), lambda i: (0,)),
        ],
        out_shape=[
            jax.ShapeDtypeStruct((N, D), jnp.bfloat16),
            jax.ShapeDtypeStruct((E,), jnp.int32),
            jax.ShapeDtypeStruct((E,), jnp.int32),
        ],
    )(x, w1t, edge_index)


def _zproj_body(x_ref, w2_ref, z_ref):
    z_ref[...] = jnp.dot(x_ref[...], w2_ref[...],
                         preferred_element_type=jnp.float32)


def _zproject(x, w2t, blk=1000):
    return pl.pallas_call(
        _zproj_body,
        grid=(N // blk,),
        in_specs=[
            pl.BlockSpec((blk, D), lambda i: (i, 0)),
            pl.BlockSpec((D, D), lambda i: (0, 0)),
        ],
        out_specs=pl.BlockSpec((blk, D), lambda i: (i, 0)),
        out_shape=jax.ShapeDtypeStruct((N, D), jnp.float32),
    )(x, w2t)


def _seg_body(y_hbm, z_hbm, z1_hbm, src_hbm, dst_hbm, out_hbm, deg_hbm,
              sidx, didx, ones_v, rows, acc, dacc, gsems, ssems, dsem):
    c = lax.axis_index("c")
    s = lax.axis_index("s")
    wid = s * NC + c
    base = wid * EPW

    r0 = s * ROWS_A

    @pl.when(s < NS - 1)
    def _():
        pltpu.sync_copy(z_hbm.at[pl.ds(0, ROWS_A)], acc.at[pl.ds(r0, ROWS_A)])

    @pl.when(s == NS - 1)
    def _():
        pltpu.sync_copy(z_hbm, acc.at[pl.ds(r0, ROWS_LAST)])

    pltpu.sync_copy(z1_hbm, dacc.at[pl.ds(s * ROWS_LAST, ROWS_LAST)])

    ov = jnp.full((16,), 1.0, jnp.float32)
    for k in range(CH // 16):
        ones_v[pl.ds(k * 16, 16)] = ov
    ones_v[pl.ds(CH - 16, 16)] = ov

    pltpu.sync_copy(src_hbm.at[pl.ds(base, EPW)], sidx)
    pltpu.sync_copy(dst_hbm.at[pl.ds(base, EPW)], didx)

    plsc.subcore_barrier()

    def wait_rows(sem):
        pltpu.make_async_copy(y_hbm.at[pl.ds(0, CH)], rows[0], sem).wait()

    def issue_gather(i, b):
        pltpu.async_copy(y_hbm.at[sidx.at[pl.ds(i * CH, CH)]], rows[b], gsems[b])

    for b in range(LOOK):
        issue_gather(b, b)

    def body(i, b):
        wait_rows(gsems[b])
        didx_i = didx.at[pl.ds(i * CH, CH)]
        pltpu.async_copy(rows[b], acc.at[didx_i], ssems[b], add=True)
        pltpu.async_copy(ones_v, dacc.at[didx_i], dsem, add=True)
        j = i + LOOK
        bj = (b + LOOK) % NBUF

        @pl.when(jnp.logical_and(j < NCHUNK, i >= NBUF - LOOK))
        def _():
            wait_rows(ssems[bj])
            issue_gather(j, bj)

        @pl.when(jnp.logical_and(j < NCHUNK, i < NBUF - LOOK))
        def _():
            issue_gather(j, bj)

    def group(g, carry):
        for b in range(NBUF):
            body(g * NBUF + b, b)
        return carry

    lax.fori_loop(0, NMAIN // NBUF, group, 0)

    for i in range(NMAIN, NCHUNK):
        b = i % NBUF
        wait_rows(gsems[b])
        didx_t = didx.at[pl.ds(i * CH, CH)]
        pltpu.async_copy(rows[b], acc.at[didx_t], ssems[b], add=True)
        pltpu.async_copy(ones_v, dacc.at[didx_t], dsem, add=True)

    for b in range(NBUF):
        wait_rows(ssems[b])
    pltpu.make_async_copy(src_hbm.at[pl.ds(0, EPW)], sidx, dsem).wait()

    plsc.subcore_barrier()

    @pl.when(s < NS - 1)
    def _():
        pltpu.sync_copy(acc.at[pl.ds(r0, ROWS_A)], out_hbm.at[c, pl.ds(r0, ROWS_A)])

    @pl.when(s == NS - 1)
    def _():
        pltpu.sync_copy(acc.at[pl.ds(r0, ROWS_LAST)],
                        out_hbm.at[c, pl.ds(r0, ROWS_LAST)])

    pltpu.sync_copy(dacc.at[pl.ds(s * ROWS_LAST, ROWS_LAST)],
                    deg_hbm.at[c, pl.ds(s * ROWS_LAST, ROWS_LAST)])


def _segment_sum(y, zeros, zeros1, src, dst):
    mesh = plsc.VectorSubcoreMesh(core_axis_name="c", subcore_axis_name="s")
    f = functools.partial(
        pl.kernel,
        out_type=[
            jax.ShapeDtypeStruct((NC, N, D), jnp.bfloat16),
            jax.ShapeDtypeStruct((NC, DEG_P), jnp.float32),
        ],
        mesh=mesh,
        scratch_types=[
            pltpu.VMEM((EPW,), jnp.int32),
            pltpu.VMEM((EPW,), jnp.int32),
            pltpu.VMEM((CH,), jnp.float32),
            [pltpu.VMEM((CH, D), jnp.bfloat16) for _ in range(NBUF)],
            pltpu.VMEM_SHARED((N, D), jnp.bfloat16),
            pltpu.VMEM_SHARED((DEG_P,), jnp.float32),
            [pltpu.SemaphoreType.DMA for _ in range(NBUF)],
            [pltpu.SemaphoreType.DMA for _ in range(NBUF)],
            pltpu.SemaphoreType.DMA,
        ],
        compiler_params=pltpu.CompilerParams(use_tc_tiling_on_sc=False),
    )(_seg_body)
    return f(y, zeros, zeros1, src, dst)


def _final_body(s_ref, deg_ref, z_ref, o_ref):
    stot = (s_ref[0].astype(jnp.float32)
            + s_ref[1].astype(jnp.float32))
    deg = deg_ref[0, :N] + deg_ref[1, :N]
    deg = jnp.maximum(deg, 1.0)
    o_ref[...] = stot / deg[:, None] + z_ref[...]


def _finalize(s, degs, z):
    return pl.pallas_call(
        _final_body,
        out_shape=jax.ShapeDtypeStruct((N, D), jnp.float32),
    )(s, degs, z)


def kernel(in_features, edge_index, W1, W2, layer_id):
    y, src, dst = _project(in_features, W1.T, edge_index)
    z = _zproject(in_features, W2.T)
    zeros = jnp.zeros((ROWS_LAST, D), jnp.bfloat16)
    zeros1 = jnp.zeros((ROWS_LAST,), jnp.float32)
    s, degs = _segment_sum(y, zeros, zeros1, src, dst)
    return _finalize(s, degs, z)
